# baseline (speedup 1.0000x reference)
import functools

import jax
import jax.numpy as jnp
from jax import lax
from jax.experimental import pallas as pl
from jax.experimental.pallas import tpu as pltpu
from jax.experimental.pallas import tpu_sc as plsc

N = 10000
E = 320000
NPAD = 10240
STRIPE = NPAD // 16
CHUNK = 128
NCH = 80
EPAD = 32 * NCH * CHUNK
WB = 64
BR = 400
GRID = N // BR


def _sc_mesh():
    return plsc.VectorSubcoreMesh(core_axis_name="c", subcore_axis_name="s")


def _deg_body(dst_hbm, zeros1_hbm, out0_hbm, out1_hbm, dst_v, ones_v, acc, sem):
    ci = lax.axis_index("c")
    si = lax.axis_index("s")
    wid = ci * 16 + si
    pltpu.sync_copy(dst_hbm.at[wid], dst_v)
    for j in range(CHUNK // 16):
        ones_v[pl.ds(j * 16, 16)] = jnp.ones((16,), jnp.float32)
    row0 = si * STRIPE
    pltpu.sync_copy(zeros1_hbm, acc.at[pl.ds(row0, STRIPE)])
    plsc.subcore_barrier()

    def step(k, carry):
        pltpu.sync_copy(ones_v, acc.at[dst_v.at[k]], add=True)
        return carry

    lax.fori_loop(0, NCH, step, 0)
    plsc.subcore_barrier()

    @pl.when(ci == 0)
    def _():
        pltpu.sync_copy(acc.at[pl.ds(row0, STRIPE)],
                        out0_hbm.at[pl.ds(row0, STRIPE)])

    @pl.when(ci == 1)
    def _():
        pltpu.sync_copy(acc.at[pl.ds(row0, STRIPE)],
                        out1_hbm.at[pl.ds(row0, STRIPE)])


def _deg_call(dstp, zeros1):
    k = pl.kernel(
        _deg_body,
        out_type=[jax.ShapeDtypeStruct((NPAD,), jnp.float32),
                  jax.ShapeDtypeStruct((NPAD,), jnp.float32)],
        mesh=_sc_mesh(),
        scratch_types=[
            pltpu.VMEM((NCH, CHUNK), jnp.int32),
            pltpu.VMEM((CHUNK,), jnp.float32),
            pltpu.VMEM_SHARED((NPAD,), jnp.float32),
            pltpu.SemaphoreType.DMA,
        ],
    )
    return k(dstp, zeros1)


NBUF = 4
NGRP = NCH // NBUF


def _agg_body(nblk, gflat_hbm, srck_hbm, dst_hbm, zeros_hbm, out_hbm, *rest):
    src_v, dst_v = rest[0], rest[1]
    rows = rest[2:2 + 2 * NBUF]
    acc = rest[2 + 2 * NBUF]
    gsems = rest[3 + 2 * NBUF:3 + 4 * NBUF]
    sem_s = rest[3 + 4 * NBUF]

    ci = lax.axis_index("c")
    si = lax.axis_index("s")
    wid = ci * 16 + si
    pltpu.sync_copy(dst_hbm.at[wid], dst_v)
    row0 = si * STRIPE

    for p in range(nblk):
        pltpu.sync_copy(srck_hbm.at[p, wid], src_v)
        pltpu.sync_copy(zeros_hbm, acc.at[pl.ds(row0, STRIPE)])
        plsc.subcore_barrier()

        for par in range(2):
            for b in range(NBUF):
                pltpu.async_copy(gflat_hbm.at[src_v.at[par * NBUF + b]],
                                 rows[par * NBUF + b], gsems[par * NBUF + b])

        def step2(i, carry):
            for par in range(2):
                kb = 2 * NBUF * i + par * NBUF
                for b in range(NBUF):
                    k = kb + b
                    j = par * NBUF + b
                    pltpu.make_async_copy(gflat_hbm.at[src_v.at[k]], rows[j],
                                          gsems[j]).wait()
                    pltpu.async_copy(rows[j], acc.at[dst_v.at[k]], sem_s,
                                     add=True)
                for b in range(NBUF):
                    k = kb + b
                    j = par * NBUF + b
                    pltpu.make_async_copy(rows[j], acc.at[dst_v.at[k]],
                                          sem_s).wait()

                @pl.when(i < NGRP // 2 - 1)
                def _():
                    for b in range(NBUF):
                        j = par * NBUF + b
                        pltpu.async_copy(
                            gflat_hbm.at[src_v.at[kb + 2 * NBUF + b]],
                            rows[j], gsems[j])

            return carry

        lax.fori_loop(0, NGRP // 2, step2, 0)
        plsc.subcore_barrier()
        pltpu.sync_copy(acc.at[pl.ds(row0, STRIPE)],
                        out_hbm.at[ci, pl.ds(row0, STRIPE), p])
        plsc.subcore_barrier()


def _agg_call(g, srck, dstp, zeros, nblk):
    gflat = g.reshape(nblk * N, WB)
    k = pl.kernel(
        functools.partial(_agg_body, nblk),
        out_type=jax.ShapeDtypeStruct((2, NPAD, nblk, WB), jnp.float32),
        mesh=_sc_mesh(),
        scratch_types=(
            [pltpu.VMEM((NCH, CHUNK), jnp.int32),
             pltpu.VMEM((NCH, CHUNK), jnp.int32)]
            + [pltpu.VMEM((CHUNK, WB), jnp.float32)] * (2 * NBUF)
            + [pltpu.VMEM_SHARED((NPAD, WB), jnp.float32)]
            + [pltpu.SemaphoreType.DMA] * (2 * NBUF)
            + [pltpu.SemaphoreType.DMA]
        ),
        compiler_params=pltpu.CompilerParams(use_tc_tiling_on_sc=False),
    )
    out = k(gflat, srck, dstp, zeros)
    return out.reshape(2, NPAD, nblk * WB)


def _prep_body(s_ref, x_ref, g0_ref):
    g0_ref[...] = x_ref[...] * s_ref[...]


def _prep_call(s, x):
    return pl.pallas_call(
        _prep_body,
        grid=(GRID,),
        in_specs=[
            pl.BlockSpec((BR, 1), lambda i: (i, 0)),
            pl.BlockSpec((BR, 128), lambda i: (i, 0)),
        ],
        out_specs=pl.BlockSpec((BR, 128), lambda i: (i, 0)),
        out_shape=jax.ShapeDtypeStruct((N, 128), jnp.float32),
    )(s, x)


def _l0_body(e0_ref, g0_ref, s_ref, W0_ref, b0_ref, g1_ref):
    s = s_ref[...]
    agg = s * (e0_ref[0] + e0_ref[1] + g0_ref[...])
    h = jnp.dot(agg.astype(jnp.bfloat16), W0_ref[...].astype(jnp.bfloat16),
                preferred_element_type=jnp.float32)
    h = jnp.maximum(h + b0_ref[...], 0.0)
    g1_ref[...] = s * h


def _l0_call(e0, g0, s, W0, b0):
    return pl.pallas_call(
        _l0_body,
        grid=(GRID,),
        in_specs=[
            pl.BlockSpec((2, BR, 128), lambda i: (0, i, 0)),
            pl.BlockSpec((BR, 128), lambda i: (i, 0)),
            pl.BlockSpec((BR, 1), lambda i: (i, 0)),
            pl.BlockSpec((128, 256), lambda i: (0, 0)),
            pl.BlockSpec((1, 256), lambda i: (0, 0)),
        ],
        out_specs=pl.BlockSpec((BR, 256), lambda i: (i, 0)),
        out_shape=jax.ShapeDtypeStruct((N, 256), jnp.float32),
    )(e0, g0, s, W0, b0)


def _l1_body(e1_ref, g1_ref, s_ref, W1_ref, b1_ref, W2_ref, q_ref):
    s = s_ref[...]
    agg = s * (e1_ref[0] + e1_ref[1] + g1_ref[...])
    h = jnp.dot(agg.astype(jnp.bfloat16), W1_ref[...].astype(jnp.bfloat16),
                preferred_element_type=jnp.float32)
    h = jnp.maximum(h + b1_ref[...], 0.0)
    q_ref[...] = s * jnp.dot(h.astype(jnp.bfloat16),
                             W2_ref[...].astype(jnp.bfloat16),
                             preferred_element_type=jnp.float32)


def _l1_call(e1, g1, s, W1, b1, W2p):
    return pl.pallas_call(
        _l1_body,
        grid=(GRID,),
        in_specs=[
            pl.BlockSpec((2, BR, 256), lambda i: (0, i, 0)),
            pl.BlockSpec((BR, 256), lambda i: (i, 0)),
            pl.BlockSpec((BR, 1), lambda i: (i, 0)),
            pl.BlockSpec((256, 256), lambda i: (0, 0)),
            pl.BlockSpec((1, 256), lambda i: (0, 0)),
            pl.BlockSpec((256, WB), lambda i: (0, 0)),
        ],
        out_specs=pl.BlockSpec((BR, WB), lambda i: (i, 0)),
        out_shape=jax.ShapeDtypeStruct((N, WB), jnp.float32),
    )(e1, g1, s, W1, b1, W2p)


def _fin_body(e2_ref, q_ref, s_ref, b2_ref, out_ref):
    z = s_ref[...] * (e2_ref[0] + e2_ref[1] + q_ref[...]) + b2_ref[...]
    z = z[:, :40]
    m = jnp.max(z, axis=1, keepdims=True)
    ez = jnp.exp(z - m)
    lse = jnp.log(jnp.sum(ez, axis=1, keepdims=True)) + m
    out_ref[...] = z - lse


def _fin_call(e2, q, s, b2p):
    return pl.pallas_call(
        _fin_body,
        grid=(GRID,),
        in_specs=[
            pl.BlockSpec((2, BR, WB), lambda i: (0, i, 0)),
            pl.BlockSpec((BR, WB), lambda i: (i, 0)),
            pl.BlockSpec((BR, 1), lambda i: (i, 0)),
            pl.BlockSpec((1, WB), lambda i: (0, 0)),
        ],
        out_specs=pl.BlockSpec((BR, 40), lambda i: (i, 0)),
        out_shape=jax.ShapeDtypeStruct((N, 40), jnp.float32),
    )(e2, q, s, b2p)


def kernel(x, edge_index, W0, b0, W1, b1, W2, b2):
    src = edge_index[0].astype(jnp.int32)
    dst = edge_index[1].astype(jnp.int32)
    padn = EPAD - E
    ar = jnp.arange(padn, dtype=jnp.int32)
    srcp = jnp.concatenate([src, (ar * 1301) % N])
    dstp = jnp.concatenate([dst, N + ar % (NPAD - N)]).reshape(32, NCH, CHUNK)

    def srck(nblk):
        base = (nblk * srcp).reshape(1, 32, NCH, CHUNK)
        off = jnp.arange(nblk, dtype=jnp.int32).reshape(nblk, 1, 1, 1)
        return base + off

    zeros1 = jnp.zeros((STRIPE,), jnp.float32)
    zeros64 = jnp.zeros((STRIPE, WB), jnp.float32)

    degp0, degp1 = _deg_call(dstp, zeros1)
    s = lax.rsqrt(degp0[:N] + degp1[:N] + 1.0).reshape(N, 1)

    g0 = _prep_call(s, x)
    e0 = _agg_call(g0, srck(2), dstp, zeros64, 2)
    g1 = _l0_call(e0, g0, s, W0, b0.reshape(1, 256))

    e1 = _agg_call(g1, srck(4), dstp, zeros64, 4)
    W2p = jnp.pad(W2, ((0, 0), (0, WB - 40)))
    q = _l1_call(e1, g1, s, W1, b1.reshape(1, 256), W2p)

    e2 = _agg_call(q, srck(1), dstp, zeros64, 1)
    b2p = jnp.pad(b2, (0, WB - 40)).reshape(1, WB)
    return _fin_call(e2, q, s, b2p)


# speedup vs baseline: 1.0024x; 1.0024x over previous
import functools

import jax
import jax.numpy as jnp
from jax import lax
from jax.experimental import pallas as pl
from jax.experimental.pallas import tpu as pltpu
from jax.experimental.pallas import tpu_sc as plsc

N = 10000
E = 320000
NPAD = 10240
STRIPE = NPAD // 16
CHUNK = 128
NCH = 80
EPAD = 32 * NCH * CHUNK
WB = 64
BR = 400
GRID = N // BR


def _sc_mesh():
    return plsc.VectorSubcoreMesh(core_axis_name="c", subcore_axis_name="s")


def _deg_body(dst_hbm, zeros1_hbm, out0_hbm, out1_hbm, dst_v, ones_v, acc, sem):
    ci = lax.axis_index("c")
    si = lax.axis_index("s")
    wid = ci * 16 + si
    pltpu.sync_copy(dst_hbm.at[wid], dst_v)
    for j in range(CHUNK // 16):
        ones_v[pl.ds(j * 16, 16)] = jnp.ones((16,), jnp.float32)
    row0 = si * STRIPE
    pltpu.sync_copy(zeros1_hbm, acc.at[pl.ds(row0, STRIPE)])
    plsc.subcore_barrier()

    def step(k, carry):
        pltpu.sync_copy(ones_v, acc.at[dst_v.at[k]], add=True)
        return carry

    lax.fori_loop(0, NCH, step, 0)
    plsc.subcore_barrier()

    @pl.when(ci == 0)
    def _():
        pltpu.sync_copy(acc.at[pl.ds(row0, STRIPE)],
                        out0_hbm.at[pl.ds(row0, STRIPE)])

    @pl.when(ci == 1)
    def _():
        pltpu.sync_copy(acc.at[pl.ds(row0, STRIPE)],
                        out1_hbm.at[pl.ds(row0, STRIPE)])


def _deg_call(dstp, zeros1):
    k = pl.kernel(
        _deg_body,
        out_type=[jax.ShapeDtypeStruct((NPAD,), jnp.float32),
                  jax.ShapeDtypeStruct((NPAD,), jnp.float32)],
        mesh=_sc_mesh(),
        scratch_types=[
            pltpu.VMEM((NCH, CHUNK), jnp.int32),
            pltpu.VMEM((CHUNK,), jnp.float32),
            pltpu.VMEM_SHARED((NPAD,), jnp.float32),
            pltpu.SemaphoreType.DMA,
        ],
    )
    return k(dstp, zeros1)


NBUF = 4
NGRP = NCH // NBUF


def _agg_body(nblk, *refs):
    gs = refs[:nblk]
    src_hbm, dst_hbm, zeros_hbm = refs[nblk:nblk + 3]
    outs = refs[nblk + 3:2 * nblk + 3]
    rest = refs[2 * nblk + 3:]
    src_v, dst_v = rest[0], rest[1]
    rows = rest[2:2 + 2 * NBUF]
    acc = rest[2 + 2 * NBUF]
    gsems = rest[3 + 2 * NBUF:3 + 4 * NBUF]
    sem_s = rest[3 + 4 * NBUF]

    ci = lax.axis_index("c")
    si = lax.axis_index("s")
    wid = ci * 16 + si
    pltpu.sync_copy(src_hbm.at[wid], src_v)
    pltpu.sync_copy(dst_hbm.at[wid], dst_v)
    row0 = si * STRIPE

    for p in range(nblk):
        g_hbm = gs[p]
        pltpu.sync_copy(zeros_hbm, acc.at[pl.ds(row0, STRIPE)])
        plsc.subcore_barrier()

        for par in range(2):
            for b in range(NBUF):
                pltpu.async_copy(g_hbm.at[src_v.at[par * NBUF + b]],
                                 rows[par * NBUF + b], gsems[par * NBUF + b])

        def step2(i, carry):
            for par in range(2):
                kb = 2 * NBUF * i + par * NBUF
                for b in range(NBUF):
                    k = kb + b
                    j = par * NBUF + b
                    pltpu.make_async_copy(g_hbm.at[src_v.at[k]], rows[j],
                                          gsems[j]).wait()
                    pltpu.async_copy(rows[j], acc.at[dst_v.at[k]], sem_s,
                                     add=True)
                for b in range(NBUF):
                    k = kb + b
                    j = par * NBUF + b
                    pltpu.make_async_copy(rows[j], acc.at[dst_v.at[k]],
                                          sem_s).wait()

                @pl.when(i < NGRP // 2 - 1)
                def _():
                    for b in range(NBUF):
                        j = par * NBUF + b
                        pltpu.async_copy(
                            g_hbm.at[src_v.at[kb + 2 * NBUF + b]],
                            rows[j], gsems[j])

            return carry

        lax.fori_loop(0, NGRP // 2, step2, 0)
        plsc.subcore_barrier()
        pltpu.sync_copy(acc.at[pl.ds(row0, STRIPE)],
                        outs[p].at[ci, pl.ds(row0, STRIPE)])
        plsc.subcore_barrier()


def _agg_call(gs, srcp, dstp, zeros):
    nblk = len(gs)
    k = pl.kernel(
        functools.partial(_agg_body, nblk),
        out_type=[jax.ShapeDtypeStruct((2, NPAD, WB), jnp.float32)
                  for _ in range(nblk)],
        mesh=_sc_mesh(),
        scratch_types=(
            [pltpu.VMEM((NCH, CHUNK), jnp.int32),
             pltpu.VMEM((NCH, CHUNK), jnp.int32)]
            + [pltpu.VMEM((CHUNK, WB), jnp.float32)] * (2 * NBUF)
            + [pltpu.VMEM_SHARED((NPAD, WB), jnp.float32)]
            + [pltpu.SemaphoreType.DMA] * (2 * NBUF)
            + [pltpu.SemaphoreType.DMA]
        ),
        compiler_params=pltpu.CompilerParams(use_tc_tiling_on_sc=False),
    )
    return k(*gs, srcp, dstp, zeros)


def _prep_body(s_ref, x_ref, g0a_ref, g0b_ref):
    g0 = x_ref[...] * s_ref[...]
    g0a_ref[...] = g0[:, :WB]
    g0b_ref[...] = g0[:, WB:]


def _prep_call(s, x):
    return pl.pallas_call(
        _prep_body,
        grid=(GRID,),
        in_specs=[
            pl.BlockSpec((BR, 1), lambda i: (i, 0)),
            pl.BlockSpec((BR, 128), lambda i: (i, 0)),
        ],
        out_specs=[pl.BlockSpec((BR, WB), lambda i: (i, 0))] * 2,
        out_shape=[jax.ShapeDtypeStruct((N, WB), jnp.float32)] * 2,
    )(s, x)


def _pmm_body(nin, has_acc, *refs):
    e = refs[0:nin]
    g = refs[nin:2 * nin]
    if has_acc:
        acc_ref, s_ref, W_ref, o_ref = refs[2 * nin:]
    else:
        s_ref, W_ref, o_ref = refs[2 * nin:]
        acc_ref = None
    s = s_ref[...]
    cols = [e[j][0] + e[j][1] + g[j][...] for j in range(nin)]
    agg = s * (jnp.concatenate(cols, axis=1) if nin > 1 else cols[0])
    out = jnp.dot(agg.astype(jnp.bfloat16), W_ref[...].astype(jnp.bfloat16),
                  preferred_element_type=jnp.float32)
    if has_acc:
        out = out + acc_ref[...]
    o_ref[...] = out


def _pmm_call(e, g, s, W, acc=None):
    nin = len(e)
    nout = W.shape[1]
    in_specs = (
        [pl.BlockSpec((2, BR, WB), lambda i: (0, i, 0))] * nin
        + [pl.BlockSpec((BR, WB), lambda i: (i, 0))] * nin
    )
    args = list(e) + list(g)
    if acc is not None:
        in_specs.append(pl.BlockSpec((BR, nout), lambda i: (i, 0)))
        args.append(acc)
    in_specs += [
        pl.BlockSpec((BR, 1), lambda i: (i, 0)),
        pl.BlockSpec((nin * WB, nout), lambda i: (0, 0)),
    ]
    args += [s, W]
    return pl.pallas_call(
        functools.partial(_pmm_body, nin, acc is not None),
        grid=(GRID,),
        in_specs=in_specs,
        out_specs=pl.BlockSpec((BR, nout), lambda i: (i, 0)),
        out_shape=jax.ShapeDtypeStruct((N, nout), jnp.float32),
    )(*args)


def _l0fin_body(p_ref, s_ref, b0_ref, *g1_refs):
    h = jnp.maximum(p_ref[...] + b0_ref[...], 0.0)
    g1 = s_ref[...] * h
    for j in range(4):
        g1_refs[j][...] = g1[:, j * WB:(j + 1) * WB]


def _l0fin_call(p, s, b0):
    return pl.pallas_call(
        _l0fin_body,
        grid=(GRID,),
        in_specs=[
            pl.BlockSpec((BR, 256), lambda i: (i, 0)),
            pl.BlockSpec((BR, 1), lambda i: (i, 0)),
            pl.BlockSpec((1, 256), lambda i: (0, 0)),
        ],
        out_specs=[pl.BlockSpec((BR, WB), lambda i: (i, 0))] * 4,
        out_shape=[jax.ShapeDtypeStruct((N, WB), jnp.float32)] * 4,
    )(p, s, b0)


def _l1fin_body(p_ref, s_ref, b1_ref, W2_ref, q_ref):
    h = jnp.maximum(p_ref[...] + b1_ref[...], 0.0)
    q_ref[...] = s_ref[...] * jnp.dot(
        h.astype(jnp.bfloat16), W2_ref[...].astype(jnp.bfloat16),
        preferred_element_type=jnp.float32)


def _l1fin_call(p, s, b1, W2p):
    return pl.pallas_call(
        _l1fin_body,
        grid=(GRID,),
        in_specs=[
            pl.BlockSpec((BR, 256), lambda i: (i, 0)),
            pl.BlockSpec((BR, 1), lambda i: (i, 0)),
            pl.BlockSpec((1, 256), lambda i: (0, 0)),
            pl.BlockSpec((256, WB), lambda i: (0, 0)),
        ],
        out_specs=pl.BlockSpec((BR, WB), lambda i: (i, 0)),
        out_shape=jax.ShapeDtypeStruct((N, WB), jnp.float32),
    )(p, s, b1, W2p)


def _fin_body(e2_ref, q_ref, s_ref, b2_ref, out_ref):
    z = s_ref[...] * (e2_ref[0] + e2_ref[1] + q_ref[...]) + b2_ref[...]
    z = z[:, :40]
    m = jnp.max(z, axis=1, keepdims=True)
    ez = jnp.exp(z - m)
    lse = jnp.log(jnp.sum(ez, axis=1, keepdims=True)) + m
    out_ref[...] = z - lse


def _fin_call(e2, q, s, b2p):
    return pl.pallas_call(
        _fin_body,
        grid=(GRID,),
        in_specs=[
            pl.BlockSpec((2, BR, WB), lambda i: (0, i, 0)),
            pl.BlockSpec((BR, WB), lambda i: (i, 0)),
            pl.BlockSpec((BR, 1), lambda i: (i, 0)),
            pl.BlockSpec((1, WB), lambda i: (0, 0)),
        ],
        out_specs=pl.BlockSpec((BR, 40), lambda i: (i, 0)),
        out_shape=jax.ShapeDtypeStruct((N, 40), jnp.float32),
    )(e2, q, s, b2p)


def kernel(x, edge_index, W0, b0, W1, b1, W2, b2):
    src = edge_index[0].astype(jnp.int32)
    dst = edge_index[1].astype(jnp.int32)
    padn = EPAD - E
    ar = jnp.arange(padn, dtype=jnp.int32)
    srcp = jnp.concatenate([src, (ar * 1301) % N]).reshape(32, NCH, CHUNK)
    dstp = jnp.concatenate([dst, N + ar % (NPAD - N)]).reshape(32, NCH, CHUNK)

    zeros1 = jnp.zeros((STRIPE,), jnp.float32)
    zeros64 = jnp.zeros((STRIPE, WB), jnp.float32)

    degp0, degp1 = _deg_call(dstp, zeros1)
    s = lax.rsqrt(degp0[:N] + degp1[:N] + 1.0).reshape(N, 1)

    g0a, g0b = _prep_call(s, x)
    (e0a,) = _agg_call([g0a], srcp, dstp, zeros64)
    (e0b,) = _agg_call([g0b], srcp, dstp, zeros64)
    p0 = _pmm_call([e0a], [g0a], s, W0[:WB])
    p0 = _pmm_call([e0b], [g0b], s, W0[WB:], acc=p0)
    g1 = _l0fin_call(p0, s, b0.reshape(1, 256))

    e1a = _agg_call([g1[0], g1[1]], srcp, dstp, zeros64)
    e1b = _agg_call([g1[2], g1[3]], srcp, dstp, zeros64)
    p1 = _pmm_call(list(e1a), [g1[0], g1[1]], s, W1[:2 * WB])
    p1 = _pmm_call(list(e1b), [g1[2], g1[3]], s, W1[2 * WB:], acc=p1)
    W2p = jnp.pad(W2, ((0, 0), (0, WB - 40)))
    q = _l1fin_call(p1, s, b1.reshape(1, 256), W2p)

    (e2,) = _agg_call([q], srcp, dstp, zeros64)
    b2p = jnp.pad(b2, (0, WB - 40)).reshape(1, WB)
    return _fin_call(e2, q, s, b2p)


# speedup vs baseline: 1.1379x; 1.1352x over previous
import functools

import jax
import jax.numpy as jnp
from jax import lax
from jax.experimental import pallas as pl
from jax.experimental.pallas import tpu as pltpu
from jax.experimental.pallas import tpu_sc as plsc

N = 10000
E = 320000
NPAD = 10240
STRIPE = NPAD // 16
CHUNK = 128
NCH = 80
EPAD = 32 * NCH * CHUNK
WB = 64
BR = 400
GRID = N // BR


def _sc_mesh():
    return plsc.VectorSubcoreMesh(core_axis_name="c", subcore_axis_name="s")


def _deg_body(dst_hbm, zeros1_hbm, out0_hbm, out1_hbm, dst_v, ones_v, acc, sem):
    ci = lax.axis_index("c")
    si = lax.axis_index("s")
    wid = ci * 16 + si
    pltpu.sync_copy(dst_hbm.at[wid], dst_v)
    for j in range(CHUNK // 16):
        ones_v[pl.ds(j * 16, 16)] = jnp.ones((16,), jnp.float32)
    row0 = si * STRIPE
    pltpu.sync_copy(zeros1_hbm, acc.at[pl.ds(row0, STRIPE)])
    plsc.subcore_barrier()

    def step(k, carry):
        pltpu.sync_copy(ones_v, acc.at[dst_v.at[k]], add=True)
        return carry

    lax.fori_loop(0, NCH, step, 0)
    plsc.subcore_barrier()

    @pl.when(ci == 0)
    def _():
        pltpu.sync_copy(acc.at[pl.ds(row0, STRIPE)],
                        out0_hbm.at[pl.ds(row0, STRIPE)])

    @pl.when(ci == 1)
    def _():
        pltpu.sync_copy(acc.at[pl.ds(row0, STRIPE)],
                        out1_hbm.at[pl.ds(row0, STRIPE)])


def _deg_call(dstp, zeros1):
    k = pl.kernel(
        _deg_body,
        out_type=[jax.ShapeDtypeStruct((NPAD,), jnp.float32),
                  jax.ShapeDtypeStruct((NPAD,), jnp.float32)],
        mesh=_sc_mesh(),
        scratch_types=[
            pltpu.VMEM((NCH, CHUNK), jnp.int32),
            pltpu.VMEM((CHUNK,), jnp.float32),
            pltpu.VMEM_SHARED((NPAD,), jnp.float32),
            pltpu.SemaphoreType.DMA,
        ],
    )
    return k(dstp, zeros1)


NBUF = 4
NGRP = NCH // NBUF


def _agg_body(nblk, *refs):
    gs = refs[:nblk]
    src_hbm, dst_hbm, zeros_hbm = refs[nblk:nblk + 3]
    outs = refs[nblk + 3:2 * nblk + 3]
    rest = refs[2 * nblk + 3:]
    src_v, dst_v = rest[0], rest[1]
    rows = rest[2:2 + 2 * NBUF]
    acc = rest[2 + 2 * NBUF]
    gsems = rest[3 + 2 * NBUF:3 + 4 * NBUF]
    sem_s = rest[3 + 4 * NBUF]

    ci = lax.axis_index("c")
    si = lax.axis_index("s")
    wid = ci * 16 + si
    pltpu.sync_copy(src_hbm.at[wid], src_v)
    pltpu.sync_copy(dst_hbm.at[wid], dst_v)
    row0 = si * STRIPE

    for p in range(nblk):
        g_hbm = gs[p]
        pltpu.sync_copy(zeros_hbm, acc.at[pl.ds(row0, STRIPE)])
        plsc.subcore_barrier()

        for par in range(2):
            for b in range(NBUF):
                pltpu.async_copy(g_hbm.at[src_v.at[par * NBUF + b]],
                                 rows[par * NBUF + b], gsems[par * NBUF + b])

        def step2(i, carry):
            for par in range(2):
                kb = 2 * NBUF * i + par * NBUF
                for b in range(NBUF):
                    k = kb + b
                    j = par * NBUF + b
                    pltpu.make_async_copy(g_hbm.at[src_v.at[k]], rows[j],
                                          gsems[j]).wait()
                    pltpu.async_copy(rows[j], acc.at[dst_v.at[k]], sem_s,
                                     add=True)
                for b in range(NBUF):
                    k = kb + b
                    j = par * NBUF + b
                    pltpu.make_async_copy(rows[j], acc.at[dst_v.at[k]],
                                          sem_s).wait()

                @pl.when(i < NGRP // 2 - 1)
                def _():
                    for b in range(NBUF):
                        j = par * NBUF + b
                        pltpu.async_copy(
                            g_hbm.at[src_v.at[kb + 2 * NBUF + b]],
                            rows[j], gsems[j])

            return carry

        lax.fori_loop(0, NGRP // 2, step2, 0)
        plsc.subcore_barrier()
        pltpu.sync_copy(acc.at[pl.ds(row0, STRIPE)],
                        outs[p].at[ci, pl.ds(row0, STRIPE)])
        plsc.subcore_barrier()


def _agg_call(gs, srcp, dstp, zeros):
    nblk = len(gs)
    k = pl.kernel(
        functools.partial(_agg_body, nblk),
        out_type=[jax.ShapeDtypeStruct((2, NPAD, WB), jnp.float32)
                  for _ in range(nblk)],
        mesh=_sc_mesh(),
        scratch_types=(
            [pltpu.VMEM((NCH, CHUNK), jnp.int32),
             pltpu.VMEM((NCH, CHUNK), jnp.int32)]
            + [pltpu.VMEM((CHUNK, WB), jnp.float32)] * (2 * NBUF)
            + [pltpu.VMEM_SHARED((NPAD, WB), jnp.float32)]
            + [pltpu.SemaphoreType.DMA] * (2 * NBUF)
            + [pltpu.SemaphoreType.DMA]
        ),
        compiler_params=pltpu.CompilerParams(use_tc_tiling_on_sc=False),
    )
    return k(*gs, srcp, dstp, zeros)


def _prep_body(s_ref, x_ref, g0a_ref, g0b_ref):
    g0 = x_ref[...] * s_ref[...]
    g0a_ref[...] = g0[:, :WB]
    g0b_ref[...] = g0[:, WB:]


def _prep_call(s, x):
    return pl.pallas_call(
        _prep_body,
        grid=(GRID,),
        in_specs=[
            pl.BlockSpec((BR, 1), lambda i: (i, 0)),
            pl.BlockSpec((BR, 128), lambda i: (i, 0)),
        ],
        out_specs=[pl.BlockSpec((BR, WB), lambda i: (i, 0))] * 2,
        out_shape=[jax.ShapeDtypeStruct((N, WB), jnp.float32)] * 2,
    )(s, x)


def _l0_body(e0a_ref, e0b_ref, g0a_ref, g0b_ref, s_ref, W0_ref, b0_ref,
             *g1_refs):
    s = s_ref[...]
    ea = e0a_ref[0] + e0a_ref[1] + g0a_ref[...]
    eb = e0b_ref[0] + e0b_ref[1] + g0b_ref[...]
    agg = s * jnp.concatenate([ea, eb], axis=1)
    h = jnp.dot(agg.astype(jnp.bfloat16), W0_ref[...].astype(jnp.bfloat16),
                preferred_element_type=jnp.float32)
    h = jnp.maximum(h + b0_ref[...], 0.0)
    g1 = s * h
    for j in range(4):
        g1_refs[j][...] = g1[:, j * WB:(j + 1) * WB]


def _l0_call(e0a, e0b, g0a, g0b, s, W0, b0):
    return pl.pallas_call(
        _l0_body,
        grid=(GRID,),
        in_specs=[
            pl.BlockSpec((2, BR, WB), lambda i: (0, i, 0)),
            pl.BlockSpec((2, BR, WB), lambda i: (0, i, 0)),
            pl.BlockSpec((BR, WB), lambda i: (i, 0)),
            pl.BlockSpec((BR, WB), lambda i: (i, 0)),
            pl.BlockSpec((BR, 1), lambda i: (i, 0)),
            pl.BlockSpec((128, 256), lambda i: (0, 0)),
            pl.BlockSpec((1, 256), lambda i: (0, 0)),
        ],
        out_specs=[pl.BlockSpec((BR, WB), lambda i: (i, 0))] * 4,
        out_shape=[jax.ShapeDtypeStruct((N, WB), jnp.float32)] * 4,
    )(e0a, e0b, g0a, g0b, s, W0, b0)


def _l1_body(*refs):
    e1 = refs[0:4]
    g1 = refs[4:8]
    s_ref, W1_ref, b1_ref, W2_ref, q_ref = refs[8:]
    s = s_ref[...]
    cols = [e1[j][0] + e1[j][1] + g1[j][...] for j in range(4)]
    agg = s * jnp.concatenate(cols, axis=1)
    h = jnp.dot(agg.astype(jnp.bfloat16), W1_ref[...].astype(jnp.bfloat16),
                preferred_element_type=jnp.float32)
    h = jnp.maximum(h + b1_ref[...], 0.0)
    q_ref[...] = s * jnp.dot(h.astype(jnp.bfloat16),
                             W2_ref[...].astype(jnp.bfloat16),
                             preferred_element_type=jnp.float32)


def _l1_call(e1, g1, s, W1, b1, W2p):
    return pl.pallas_call(
        _l1_body,
        grid=(GRID,),
        in_specs=(
            [pl.BlockSpec((2, BR, WB), lambda i: (0, i, 0))] * 4
            + [pl.BlockSpec((BR, WB), lambda i: (i, 0))] * 4
            + [
                pl.BlockSpec((BR, 1), lambda i: (i, 0)),
                pl.BlockSpec((256, 256), lambda i: (0, 0)),
                pl.BlockSpec((1, 256), lambda i: (0, 0)),
                pl.BlockSpec((256, WB), lambda i: (0, 0)),
            ]
        ),
        out_specs=pl.BlockSpec((BR, WB), lambda i: (i, 0)),
        out_shape=jax.ShapeDtypeStruct((N, WB), jnp.float32),
    )(*e1, *g1, s, W1, b1, W2p)


def _fin_body(e2_ref, q_ref, s_ref, b2_ref, out_ref):
    z = s_ref[...] * (e2_ref[0] + e2_ref[1] + q_ref[...]) + b2_ref[...]
    z = z[:, :40]
    m = jnp.max(z, axis=1, keepdims=True)
    ez = jnp.exp(z - m)
    lse = jnp.log(jnp.sum(ez, axis=1, keepdims=True)) + m
    out_ref[...] = z - lse


def _fin_call(e2, q, s, b2p):
    return pl.pallas_call(
        _fin_body,
        grid=(GRID,),
        in_specs=[
            pl.BlockSpec((2, BR, WB), lambda i: (0, i, 0)),
            pl.BlockSpec((BR, WB), lambda i: (i, 0)),
            pl.BlockSpec((BR, 1), lambda i: (i, 0)),
            pl.BlockSpec((1, WB), lambda i: (0, 0)),
        ],
        out_specs=pl.BlockSpec((BR, 40), lambda i: (i, 0)),
        out_shape=jax.ShapeDtypeStruct((N, 40), jnp.float32),
    )(e2, q, s, b2p)


def kernel(x, edge_index, W0, b0, W1, b1, W2, b2):
    src = edge_index[0].astype(jnp.int32)
    dst = edge_index[1].astype(jnp.int32)
    padn = EPAD - E
    ar = jnp.arange(padn, dtype=jnp.int32)
    srcp = jnp.concatenate([src, (ar * 1301) % N]).reshape(32, NCH, CHUNK)
    dstp = jnp.concatenate([dst, N + ar % (NPAD - N)]).reshape(32, NCH, CHUNK)

    zeros1 = jnp.zeros((STRIPE,), jnp.float32)
    zeros64 = jnp.zeros((STRIPE, WB), jnp.float32)

    degp0, degp1 = _deg_call(dstp, zeros1)
    s = lax.rsqrt(degp0[:N] + degp1[:N] + 1.0).reshape(N, 1)

    g0a, g0b = _prep_call(s, x)
    e0a, e0b = _agg_call([g0a, g0b], srcp, dstp, zeros64)
    g1 = _l0_call(e0a, e0b, g0a, g0b, s, W0, b0.reshape(1, 256))

    e1 = _agg_call(list(g1), srcp, dstp, zeros64)
    W2p = jnp.pad(W2, ((0, 0), (0, WB - 40)))
    q = _l1_call(list(e1), list(g1), s, W1, b1.reshape(1, 256), W2p)

    (e2,) = _agg_call([q], srcp, dstp, zeros64)
    b2p = jnp.pad(b2, (0, WB - 40)).reshape(1, WB)
    return _fin_call(e2, q, s, b2p)


# speedup vs baseline: 1.2088x; 1.0623x over previous
import functools

import jax
import jax.numpy as jnp
from jax import lax
from jax.experimental import pallas as pl
from jax.experimental.pallas import tpu as pltpu
from jax.experimental.pallas import tpu_sc as plsc

N = 10000
E = 320000
NPAD = 10240
STRIPE = NPAD // 16
CHUNK = 128
NCH = 80
EPAD = 32 * NCH * CHUNK
WB = 64
BR = 2000
GRID = N // BR


def _sc_mesh():
    return plsc.VectorSubcoreMesh(core_axis_name="c", subcore_axis_name="s")


def _deg_body(dst_hbm, zeros1_hbm, out0_hbm, out1_hbm, dst_v, ones_v, acc, sem):
    ci = lax.axis_index("c")
    si = lax.axis_index("s")
    wid = ci * 16 + si
    pltpu.sync_copy(dst_hbm.at[wid], dst_v)
    for j in range(CHUNK // 16):
        ones_v[pl.ds(j * 16, 16)] = jnp.ones((16,), jnp.float32)
    row0 = si * STRIPE
    pltpu.sync_copy(zeros1_hbm, acc.at[pl.ds(row0, STRIPE)])
    plsc.subcore_barrier()

    def step(k, carry):
        pltpu.sync_copy(ones_v, acc.at[dst_v.at[k]], add=True)
        return carry

    lax.fori_loop(0, NCH, step, 0)
    plsc.subcore_barrier()

    @pl.when(ci == 0)
    def _():
        pltpu.sync_copy(acc.at[pl.ds(row0, STRIPE)],
                        out0_hbm.at[pl.ds(row0, STRIPE)])

    @pl.when(ci == 1)
    def _():
        pltpu.sync_copy(acc.at[pl.ds(row0, STRIPE)],
                        out1_hbm.at[pl.ds(row0, STRIPE)])


def _deg_call(dstp, zeros1):
    k = pl.kernel(
        _deg_body,
        out_type=[jax.ShapeDtypeStruct((NPAD,), jnp.float32),
                  jax.ShapeDtypeStruct((NPAD,), jnp.float32)],
        mesh=_sc_mesh(),
        scratch_types=[
            pltpu.VMEM((NCH, CHUNK), jnp.int32),
            pltpu.VMEM((CHUNK,), jnp.float32),
            pltpu.VMEM_SHARED((NPAD,), jnp.float32),
            pltpu.SemaphoreType.DMA,
        ],
    )
    return k(dstp, zeros1)


NBUF = 4
NGRP = NCH // NBUF


def _agg_body(nblk, *refs):
    gs = refs[:nblk]
    src_hbm, dst_hbm, zeros_hbm = refs[nblk:nblk + 3]
    outs = refs[nblk + 3:2 * nblk + 3]
    rest = refs[2 * nblk + 3:]
    src_v, dst_v = rest[0], rest[1]
    rows = rest[2:2 + 2 * NBUF]
    acc = rest[2 + 2 * NBUF]
    gsems = rest[3 + 2 * NBUF:3 + 4 * NBUF]
    sem_s = rest[3 + 4 * NBUF]

    ci = lax.axis_index("c")
    si = lax.axis_index("s")
    wid = ci * 16 + si
    pltpu.sync_copy(src_hbm.at[wid], src_v)
    pltpu.sync_copy(dst_hbm.at[wid], dst_v)
    row0 = si * STRIPE

    for p in range(nblk):
        g_hbm = gs[p]
        pltpu.sync_copy(zeros_hbm, acc.at[pl.ds(row0, STRIPE)])
        plsc.subcore_barrier()

        for par in range(2):
            for b in range(NBUF):
                pltpu.async_copy(g_hbm.at[src_v.at[par * NBUF + b]],
                                 rows[par * NBUF + b], gsems[par * NBUF + b])

        def step2(i, carry):
            for par in range(2):
                kb = 2 * NBUF * i + par * NBUF
                for b in range(NBUF):
                    k = kb + b
                    j = par * NBUF + b
                    pltpu.make_async_copy(g_hbm.at[src_v.at[k]], rows[j],
                                          gsems[j]).wait()
                    pltpu.async_copy(rows[j], acc.at[dst_v.at[k]], sem_s,
                                     add=True)
                for b in range(NBUF):
                    k = kb + b
                    j = par * NBUF + b
                    pltpu.make_async_copy(rows[j], acc.at[dst_v.at[k]],
                                          sem_s).wait()

                @pl.when(i < NGRP // 2 - 1)
                def _():
                    for b in range(NBUF):
                        j = par * NBUF + b
                        pltpu.async_copy(
                            g_hbm.at[src_v.at[kb + 2 * NBUF + b]],
                            rows[j], gsems[j])

            return carry

        lax.fori_loop(0, NGRP // 2, step2, 0)
        plsc.subcore_barrier()
        pltpu.sync_copy(acc.at[pl.ds(row0, STRIPE)],
                        outs[p].at[ci, pl.ds(row0, STRIPE)])
        plsc.subcore_barrier()


def _agg_call(gs, srcp, dstp, zeros):
    nblk = len(gs)
    k = pl.kernel(
        functools.partial(_agg_body, nblk),
        out_type=[jax.ShapeDtypeStruct((2, NPAD, WB), jnp.float32)
                  for _ in range(nblk)],
        mesh=_sc_mesh(),
        scratch_types=(
            [pltpu.VMEM((NCH, CHUNK), jnp.int32),
             pltpu.VMEM((NCH, CHUNK), jnp.int32)]
            + [pltpu.VMEM((CHUNK, WB), jnp.float32)] * (2 * NBUF)
            + [pltpu.VMEM_SHARED((NPAD, WB), jnp.float32)]
            + [pltpu.SemaphoreType.DMA] * (2 * NBUF)
            + [pltpu.SemaphoreType.DMA]
        ),
        compiler_params=pltpu.CompilerParams(use_tc_tiling_on_sc=False),
    )
    return k(*gs, srcp, dstp, zeros)


def _prep_body(s_ref, x_ref, g0a_ref, g0b_ref):
    g0 = x_ref[...] * s_ref[...]
    g0a_ref[...] = g0[:, :WB]
    g0b_ref[...] = g0[:, WB:]


def _prep_call(s, x):
    return pl.pallas_call(
        _prep_body,
        grid=(GRID,),
        in_specs=[
            pl.BlockSpec((BR, 1), lambda i: (i, 0)),
            pl.BlockSpec((BR, 128), lambda i: (i, 0)),
        ],
        out_specs=[pl.BlockSpec((BR, WB), lambda i: (i, 0))] * 2,
        out_shape=[jax.ShapeDtypeStruct((N, WB), jnp.float32)] * 2,
    )(s, x)


def _l0_body(e0a_ref, e0b_ref, g0a_ref, g0b_ref, s_ref, W0_ref, b0_ref,
             *g1_refs):
    s = s_ref[...]
    ea = e0a_ref[0] + e0a_ref[1] + g0a_ref[...]
    eb = e0b_ref[0] + e0b_ref[1] + g0b_ref[...]
    agg = s * jnp.concatenate([ea, eb], axis=1)
    h = jnp.dot(agg.astype(jnp.bfloat16), W0_ref[...].astype(jnp.bfloat16),
                preferred_element_type=jnp.float32)
    h = jnp.maximum(h + b0_ref[...], 0.0)
    g1 = s * h
    for j in range(4):
        g1_refs[j][...] = g1[:, j * WB:(j + 1) * WB]


def _l0_call(e0a, e0b, g0a, g0b, s, W0, b0):
    return pl.pallas_call(
        _l0_body,
        grid=(GRID,),
        in_specs=[
            pl.BlockSpec((2, BR, WB), lambda i: (0, i, 0)),
            pl.BlockSpec((2, BR, WB), lambda i: (0, i, 0)),
            pl.BlockSpec((BR, WB), lambda i: (i, 0)),
            pl.BlockSpec((BR, WB), lambda i: (i, 0)),
            pl.BlockSpec((BR, 1), lambda i: (i, 0)),
            pl.BlockSpec((128, 256), lambda i: (0, 0)),
            pl.BlockSpec((1, 256), lambda i: (0, 0)),
        ],
        out_specs=[pl.BlockSpec((BR, WB), lambda i: (i, 0))] * 4,
        out_shape=[jax.ShapeDtypeStruct((N, WB), jnp.float32)] * 4,
    )(e0a, e0b, g0a, g0b, s, W0, b0)


def _l1_body(*refs):
    e1 = refs[0:4]
    g1 = refs[4:8]
    s_ref, W1_ref, b1_ref, W2_ref, q_ref = refs[8:]
    s = s_ref[...]
    cols = [e1[j][0] + e1[j][1] + g1[j][...] for j in range(4)]
    agg = s * jnp.concatenate(cols, axis=1)
    h = jnp.dot(agg.astype(jnp.bfloat16), W1_ref[...].astype(jnp.bfloat16),
                preferred_element_type=jnp.float32)
    h = jnp.maximum(h + b1_ref[...], 0.0)
    q_ref[...] = s * jnp.dot(h.astype(jnp.bfloat16),
                             W2_ref[...].astype(jnp.bfloat16),
                             preferred_element_type=jnp.float32)


def _l1_call(e1, g1, s, W1, b1, W2p):
    return pl.pallas_call(
        _l1_body,
        grid=(GRID,),
        in_specs=(
            [pl.BlockSpec((2, BR, WB), lambda i: (0, i, 0))] * 4
            + [pl.BlockSpec((BR, WB), lambda i: (i, 0))] * 4
            + [
                pl.BlockSpec((BR, 1), lambda i: (i, 0)),
                pl.BlockSpec((256, 256), lambda i: (0, 0)),
                pl.BlockSpec((1, 256), lambda i: (0, 0)),
                pl.BlockSpec((256, WB), lambda i: (0, 0)),
            ]
        ),
        out_specs=pl.BlockSpec((BR, WB), lambda i: (i, 0)),
        out_shape=jax.ShapeDtypeStruct((N, WB), jnp.float32),
    )(*e1, *g1, s, W1, b1, W2p)


def _fin_body(e2_ref, q_ref, s_ref, b2_ref, out_ref):
    z = s_ref[...] * (e2_ref[0] + e2_ref[1] + q_ref[...]) + b2_ref[...]
    z = z[:, :40]
    m = jnp.max(z, axis=1, keepdims=True)
    ez = jnp.exp(z - m)
    lse = jnp.log(jnp.sum(ez, axis=1, keepdims=True)) + m
    out_ref[...] = z - lse


def _fin_call(e2, q, s, b2p):
    return pl.pallas_call(
        _fin_body,
        grid=(GRID,),
        in_specs=[
            pl.BlockSpec((2, BR, WB), lambda i: (0, i, 0)),
            pl.BlockSpec((BR, WB), lambda i: (i, 0)),
            pl.BlockSpec((BR, 1), lambda i: (i, 0)),
            pl.BlockSpec((1, WB), lambda i: (0, 0)),
        ],
        out_specs=pl.BlockSpec((BR, 40), lambda i: (i, 0)),
        out_shape=jax.ShapeDtypeStruct((N, 40), jnp.float32),
    )(e2, q, s, b2p)


def kernel(x, edge_index, W0, b0, W1, b1, W2, b2):
    src = edge_index[0].astype(jnp.int32)
    dst = edge_index[1].astype(jnp.int32)
    padn = EPAD - E
    ar = jnp.arange(padn, dtype=jnp.int32)
    srcp = jnp.concatenate([src, (ar * 1301) % N]).reshape(32, NCH, CHUNK)
    dstp = jnp.concatenate([dst, N + ar % (NPAD - N)]).reshape(32, NCH, CHUNK)

    zeros1 = jnp.zeros((STRIPE,), jnp.float32)
    zeros64 = jnp.zeros((STRIPE, WB), jnp.float32)

    degp0, degp1 = _deg_call(dstp, zeros1)
    s = lax.rsqrt(degp0[:N] + degp1[:N] + 1.0).reshape(N, 1)

    g0a, g0b = _prep_call(s, x)
    e0a, e0b = _agg_call([g0a, g0b], srcp, dstp, zeros64)
    g1 = _l0_call(e0a, e0b, g0a, g0b, s, W0, b0.reshape(1, 256))

    e1 = _agg_call(list(g1), srcp, dstp, zeros64)
    W2p = jnp.pad(W2, ((0, 0), (0, WB - 40)))
    q = _l1_call(list(e1), list(g1), s, W1, b1.reshape(1, 256), W2p)

    (e2,) = _agg_call([q], srcp, dstp, zeros64)
    b2p = jnp.pad(b2, (0, WB - 40)).reshape(1, WB)
    return _fin_call(e2, q, s, b2p)
